# baseline (speedup 1.0000x reference)
import functools

import jax
import jax.numpy as jnp
from jax.experimental import pallas as pl
from jax.experimental.pallas import tpu as pltpu


def _body(x_ref, wp_ref, wc_ref, b_ref, out_ref, *, inv_s):
    j = pl.program_id(1)

    @pl.when(j == 0)
    def _init():
        out_ref[...] = jnp.broadcast_to(b_ref[...], out_ref.shape)

    x = x_ref[...].astype(jnp.float32)
    pooled = jnp.sum(x, axis=(2, 3))
    w_comb = jnp.dot(
        wp_ref[...], wc_ref[...], preferred_element_type=jnp.float32
    ) * inv_s
    out_ref[...] += jnp.dot(pooled, w_comb, preferred_element_type=jnp.float32)


def kernel(x_nchw, w_proj, w_cls, b_cls):
    B, C, H, W = x_nchw.shape
    S = H * W
    NCLS = w_cls.shape[1]
    FD = w_proj.shape[1]

    TB = 8
    CB = 128
    bias = b_cls.astype(jnp.float32).reshape(1, NCLS)

    out = pl.pallas_call(
        functools.partial(_body, inv_s=1.0 / float(S)),
        out_shape=jax.ShapeDtypeStruct((B, NCLS), jnp.float32),
        grid=(B // TB, C // CB),
        in_specs=[
            pl.BlockSpec((TB, CB, H, W), lambda i, j: (i, j, 0, 0)),
            pl.BlockSpec((CB, FD), lambda i, j: (j, 0)),
            pl.BlockSpec((FD, NCLS), lambda i, j: (0, 0)),
            pl.BlockSpec((1, NCLS), lambda i, j: (0, 0)),
        ],
        out_specs=pl.BlockSpec((TB, NCLS), lambda i, j: (i, 0)),
        compiler_params=pltpu.CompilerParams(
            dimension_semantics=("parallel", "arbitrary"),
            vmem_limit_bytes=60 << 20,
        ),
    )(x_nchw, w_proj.astype(jnp.float32), w_cls.astype(jnp.float32), bias)
    return out


# speedup vs baseline: 18.0501x; 18.0501x over previous
import functools

import jax
import jax.numpy as jnp
from jax.experimental import pallas as pl
from jax.experimental.pallas import tpu as pltpu


def _body(x_ref, wp_ref, wc_ref, b_ref, out_ref, acc_ref, *, inv_s, nsteps):
    h = pl.program_id(0)
    part = jnp.sum(x_ref[...].astype(jnp.float32), axis=(0, 1))

    @pl.when(h == 0)
    def _init():
        acc_ref[...] = part

    @pl.when(h != 0)
    def _accum():
        acc_ref[...] += part

    @pl.when(h == nsteps - 1)
    def _finish():
        w_comb = jnp.dot(
            wp_ref[...], wc_ref[...], preferred_element_type=jnp.float32
        ) * inv_s
        logits = jnp.dot(
            acc_ref[...], w_comb, preferred_element_type=jnp.float32
        )
        out_ref[...] = logits + b_ref[...]


def kernel(x_nchw, w_proj, w_cls, b_cls):
    B, C, H, W = x_nchw.shape
    S = H * W
    NCLS = w_cls.shape[1]
    FD = w_proj.shape[1]

    xt = jnp.transpose(x_nchw, (2, 3, 0, 1))
    bias = b_cls.astype(jnp.float32).reshape(1, NCLS)

    out = pl.pallas_call(
        functools.partial(_body, inv_s=1.0 / float(S), nsteps=H),
        out_shape=jax.ShapeDtypeStruct((B, NCLS), jnp.float32),
        grid=(H,),
        in_specs=[
            pl.BlockSpec((1, W, B, C), lambda h: (h, 0, 0, 0)),
            pl.BlockSpec((C, FD), lambda h: (0, 0)),
            pl.BlockSpec((FD, NCLS), lambda h: (0, 0)),
            pl.BlockSpec((1, NCLS), lambda h: (0, 0)),
        ],
        out_specs=pl.BlockSpec((B, NCLS), lambda h: (0, 0)),
        scratch_shapes=[pltpu.VMEM((B, C), jnp.float32)],
        compiler_params=pltpu.CompilerParams(
            dimension_semantics=("arbitrary",),
            vmem_limit_bytes=48 << 20,
        ),
    )(xt, w_proj.astype(jnp.float32), w_cls.astype(jnp.float32), bias)
    return out
